# Initial kernel scaffold; baseline (speedup 1.0000x reference)
import jax
import jax.numpy as jnp
from jax.experimental import pallas as pl


def kernel(x, edge_index, batch):
    raise NotImplementedError("write your pallas kernel here")


# speedup vs baseline: 61.5051x; 61.5051x over previous
import jax
import jax.numpy as jnp
from jax.experimental import pallas as pl
from jax.experimental.pallas import tpu as pltpu

G = 8
BM = 512
BN = 512
BK = 512
CB = 128

_HP = jax.lax.Precision.HIGHEST
_DNT = (((0,), (0,)), ((), ()))
_DN = (((1,), (0,)), ((), ()))


def _ego_body(a_ik, m_kj, a_ij, deg_j, f4_o, f5_o, f6_o, acc):
    j = pl.program_id(1)
    k = pl.program_id(2)
    nk = pl.num_programs(2)

    @pl.when(k == 0)
    def _():
        acc[...] = jnp.zeros_like(acc)

    acc[...] += jax.lax.dot(a_ik[...], m_kj[...],
                            preferred_element_type=jnp.float32)

    @pl.when(k == nk - 1)
    def _():
        q = acc[...]
        aij = a_ij[...].astype(jnp.float32)
        f4p = jnp.sum(aij * q, axis=1, keepdims=True)
        b = ((aij + q) > 0.0).astype(jnp.float32)
        f5p = jnp.sum(b * deg_j[...], axis=1, keepdims=True)
        f6p = jnp.sum(b, axis=1, keepdims=True)

        @pl.when(j == 0)
        def _():
            f4_o[...] = f4p
            f5_o[...] = f5p
            f6_o[...] = f6p

        @pl.when(j != 0)
        def _():
            f4_o[...] += f4p
            f5_o[...] += f5p
            f6_o[...] += f6p


def _ego(A, M, deg_row, P):
    grid = (P // BM, P // BN, P // BK)
    out = pl.pallas_call(
        _ego_body,
        grid=grid,
        in_specs=[
            pl.BlockSpec((BM, BK), lambda i, j, k: (i, k)),
            pl.BlockSpec((BK, BN), lambda i, j, k: (k, j)),
            pl.BlockSpec((BM, BN), lambda i, j, k: (i, j)),
            pl.BlockSpec((1, BN), lambda i, j, k: (0, j)),
        ],
        out_specs=[
            pl.BlockSpec((BM, 1), lambda i, j, k: (i, 0)),
            pl.BlockSpec((BM, 1), lambda i, j, k: (i, 0)),
            pl.BlockSpec((BM, 1), lambda i, j, k: (i, 0)),
        ],
        out_shape=[jax.ShapeDtypeStruct((P, 1), jnp.float32)] * 3,
        scratch_shapes=[pltpu.VMEM((BM, BN), jnp.float32)],
        compiler_params=pltpu.CompilerParams(
            dimension_semantics=("parallel", "arbitrary", "arbitrary")),
    )(A, M, A, deg_row)
    return out


def _graph_counts(OH):
    ones_col = jnp.ones((OH.shape[0], 1), jnp.float32)
    return jax.lax.dot_general(OH, ones_col, _DNT,
                               preferred_element_type=jnp.float32,
                               precision=_HP)


def _moments_body(F_r, batch_r, mean_o, s2_o, sk_o, ku_o):
    Fv = F_r[...]
    bt = batch_r[...]
    P = Fv.shape[0]
    eps = jnp.float32(1e-4)

    gids = jax.lax.broadcasted_iota(jnp.int32, (P, G), 1)
    OH = (bt == gids).astype(jnp.float32)
    cnt1 = jnp.maximum(_graph_counts(OH), 1.0)

    sums = jax.lax.dot_general(OH, Fv, _DNT,
                               preferred_element_type=jnp.float32,
                               precision=_HP)
    mean = sums / cnt1
    cen = Fv - jax.lax.dot_general(OH, mean, _DN,
                                   preferred_element_type=jnp.float32,
                                   precision=_HP)
    cen = jnp.where(bt < G, cen, 0.0)
    c2 = cen * cen
    m2 = jax.lax.dot_general(OH, c2, _DNT,
                             preferred_element_type=jnp.float32,
                             precision=_HP) / cnt1
    m3 = jax.lax.dot_general(OH, c2 * cen, _DNT,
                             preferred_element_type=jnp.float32,
                             precision=_HP) / cnt1
    m4 = jax.lax.dot_general(OH, c2 * c2, _DNT,
                             preferred_element_type=jnp.float32,
                             precision=_HP) / cnt1

    mean_o[...] = mean
    s2_o[...] = jnp.sqrt(m2)
    sk_o[...] = m3 / jnp.maximum(m2 * jnp.sqrt(m2), eps)
    ku_o[...] = m4 / jnp.maximum(m2 * m2, eps)


def _median_body(F_r, batch_r, med_o, lo_s, hi_s, mid_s, cnt_s, cntg_s):
    t = pl.program_id(1)
    r = pl.program_id(2)
    nt = pl.num_programs(1)
    nr = pl.num_programs(2)

    topbit = jnp.uint32(0x80000000)
    allbits = jnp.uint32(0xFFFFFFFF)

    @pl.when((t == 0) & (r == 0))
    def _():
        lo_s[...] = jnp.zeros_like(lo_s)
        hi_s[...] = jnp.full_like(hi_s, allbits)
        cntg_s[...] = jnp.zeros_like(cntg_s)

    @pl.when(r == 0)
    def _():
        lo = lo_s[...]
        mid_s[...] = lo + ((hi_s[...] - lo) >> jnp.uint32(1))
        cnt_s[...] = jnp.zeros_like(cnt_s)

    Fv = F_r[...]
    bt = batch_r[...]
    BR = Fv.shape[0]
    gids = jax.lax.broadcasted_iota(jnp.int32, (BR, G), 1)
    OH = (bt == gids).astype(jnp.float32)

    @pl.when(t == 0)
    def _():
        cntg_s[...] += _graph_counts(OH)

    bits = jax.lax.bitcast_convert_type(Fv, jnp.uint32)
    u = jnp.where(bits >= topbit, bits ^ allbits, bits | topbit)

    mid = mid_s[...]
    mid_hi = (mid >> jnp.uint32(16)).astype(jnp.float32)
    mid_lo = (mid & jnp.uint32(0xFFFF)).astype(jnp.float32)
    thr_hi = jax.lax.dot_general(OH, mid_hi, _DN,
                                 preferred_element_type=jnp.float32,
                                 precision=_HP)
    thr_lo = jax.lax.dot_general(OH, mid_lo, _DN,
                                 preferred_element_type=jnp.float32,
                                 precision=_HP)
    thr = (thr_hi.astype(jnp.uint32) << jnp.uint32(16)) | \
        thr_lo.astype(jnp.uint32)
    le = (u <= thr).astype(jnp.float32)
    cnt_s[...] += jax.lax.dot_general(OH, le, _DNT,
                                      preferred_element_type=jnp.float32,
                                      precision=_HP)

    @pl.when(r == nr - 1)
    def _():
        cnt_i = cntg_s[...].astype(jnp.int32)
        need = (((cnt_i - 1) // 2) + 1).astype(jnp.float32)
        cond = cnt_s[...] >= need
        mid = mid_s[...]
        hi_s[...] = jnp.where(cond, mid, hi_s[...])
        lo_s[...] = jnp.where(cond, lo_s[...], mid + jnp.uint32(1))

        @pl.when(t == nt - 1)
        def _():
            med_u = hi_s[...]
            mbits = jnp.where(med_u >= topbit, med_u ^ topbit,
                              med_u ^ allbits)
            med_o[...] = jax.lax.bitcast_convert_type(mbits, jnp.float32)


def _stats(F, batch_col, P, CP):
    mean, s2, sk, ku = pl.pallas_call(
        _moments_body,
        grid=(CP // CB,),
        in_specs=[
            pl.BlockSpec((P, CB), lambda c: (0, c)),
            pl.BlockSpec((P, 1), lambda c: (0, 0)),
        ],
        out_specs=[pl.BlockSpec((G, CB), lambda c: (0, c))] * 4,
        out_shape=[jax.ShapeDtypeStruct((G, CP), jnp.float32)] * 4,
        compiler_params=pltpu.CompilerParams(
            dimension_semantics=("arbitrary",)),
    )(F, batch_col)
    BR = min(2048, P)
    med, = pl.pallas_call(
        _median_body,
        grid=(CP // CB, 32, P // BR),
        in_specs=[
            pl.BlockSpec((BR, CB), lambda c, t, r: (r, c)),
            pl.BlockSpec((BR, 1), lambda c, t, r: (r, 0)),
        ],
        out_specs=[pl.BlockSpec((G, CB), lambda c, t, r: (0, c))],
        out_shape=[jax.ShapeDtypeStruct((G, CP), jnp.float32)],
        scratch_shapes=[
            pltpu.VMEM((G, CB), jnp.uint32),
            pltpu.VMEM((G, CB), jnp.uint32),
            pltpu.VMEM((G, CB), jnp.uint32),
            pltpu.VMEM((G, CB), jnp.float32),
            pltpu.VMEM((G, 1), jnp.float32),
        ],
        compiler_params=pltpu.CompilerParams(
            dimension_semantics=("arbitrary", "arbitrary", "arbitrary")),
    )(F, batch_col)
    return mean, med, s2, sk, ku


def kernel(x, edge_index, batch):
    N, DX = x.shape
    P = ((N + BM - 1) // BM) * BM
    row = edge_index[0].astype(jnp.int32)
    col = edge_index[1].astype(jnp.int32)

    deg = jnp.zeros((N,), jnp.float32).at[row].add(1.0)

    Mf = jnp.zeros((P, P), jnp.float32).at[row, col].add(1.0)
    Mb = Mf.astype(jnp.bfloat16)
    idx = jnp.arange(N)
    Ab = (Mf > 0).astype(jnp.bfloat16).at[idx, idx].set(jnp.bfloat16(1))
    deg_row = jnp.zeros((1, P), jnp.float32).at[0, :N].set(deg)

    f4p, f5rp, f6rp = _ego(Ab, Mb, deg_row, P)
    f4 = f4p[:N, 0]
    f5r = f5rp[:N, 0]
    f6r = f6rp[:N, 0]

    small = deg <= 1.0
    denom = jnp.where(small, 1.0, deg * (deg - 1.0))
    f1 = jnp.where(small, 0.0, (f4 - deg) / denom)
    cmax = jnp.maximum(deg, 1.0)
    feat2 = jnp.zeros((N,), jnp.float32).at[row].add(deg[col]) / cmax
    f3 = jnp.zeros((N,), jnp.float32).at[row].add(f1[col]) / cmax
    f5f = f5r - 2.0 * f4
    f6f = f6r - deg - 1.0

    topo = jnp.stack([deg, f1, feat2, f3, f4, f5f, f6f], axis=1)
    F = jnp.concatenate([x, topo], axis=1)
    C = DX + 7
    CP = ((C + CB - 1) // CB) * CB
    Fp = jnp.zeros((P, CP), jnp.float32).at[:N, :C].set(F)
    btp = jnp.full((P, 1), 255, jnp.int32).at[:N, 0].set(batch.astype(jnp.int32))

    mean, med, s2, sk, ku = _stats(Fp, btp, P, CP)
    return jnp.hstack([mean[:, :C], med[:, :C], s2[:, :C],
                       sk[:, :C], ku[:, :C]])


# speedup vs baseline: 72.3055x; 1.1756x over previous
import functools

import jax
import jax.numpy as jnp
from jax import lax
from jax.experimental import pallas as pl
from jax.experimental.pallas import tpu as pltpu
from jax.experimental.pallas import tpu_sc as plsc

G = 8
BM = 512
BN = 512
BK = 512
CB = 128

_HP = jax.lax.Precision.HIGHEST
_DNT = (((0,), (0,)), ((), ()))
_DN = (((1,), (0,)), ((), ()))


def _ego_body(a_ik, m_kj, a_ij, deg_j, f4_o, f5_o, f6_o, acc):
    j = pl.program_id(1)
    k = pl.program_id(2)
    nk = pl.num_programs(2)

    @pl.when(k == 0)
    def _():
        acc[...] = jnp.zeros_like(acc)

    acc[...] += jax.lax.dot(a_ik[...], m_kj[...],
                            preferred_element_type=jnp.float32)

    @pl.when(k == nk - 1)
    def _():
        q = acc[...]
        aij = a_ij[...].astype(jnp.float32)
        f4p = jnp.sum(aij * q, axis=1, keepdims=True)
        b = ((aij + q) > 0.0).astype(jnp.float32)
        f5p = jnp.sum(b * deg_j[...], axis=1, keepdims=True)
        f6p = jnp.sum(b, axis=1, keepdims=True)

        @pl.when(j == 0)
        def _():
            f4_o[...] = f4p
            f5_o[...] = f5p
            f6_o[...] = f6p

        @pl.when(j != 0)
        def _():
            f4_o[...] += f4p
            f5_o[...] += f5p
            f6_o[...] += f6p


def _ego(A, M, deg_row, P):
    grid = (P // BM, P // BN, P // BK)
    out = pl.pallas_call(
        _ego_body,
        grid=grid,
        in_specs=[
            pl.BlockSpec((BM, BK), lambda i, j, k: (i, k)),
            pl.BlockSpec((BK, BN), lambda i, j, k: (k, j)),
            pl.BlockSpec((BM, BN), lambda i, j, k: (i, j)),
            pl.BlockSpec((1, BN), lambda i, j, k: (0, j)),
        ],
        out_specs=[
            pl.BlockSpec((BM, 1), lambda i, j, k: (i, 0)),
            pl.BlockSpec((BM, 1), lambda i, j, k: (i, 0)),
            pl.BlockSpec((BM, 1), lambda i, j, k: (i, 0)),
        ],
        out_shape=[jax.ShapeDtypeStruct((P, 1), jnp.float32)] * 3,
        scratch_shapes=[pltpu.VMEM((BM, BN), jnp.float32)],
        compiler_params=pltpu.CompilerParams(
            dimension_semantics=("parallel", "arbitrary", "arbitrary")),
    )(A, M, A, deg_row)
    return out


def _graph_counts(OH):
    ones_col = jnp.ones((OH.shape[0], 1), jnp.float32)
    return jax.lax.dot_general(OH, ones_col, _DNT,
                               preferred_element_type=jnp.float32,
                               precision=_HP)


def _moments_body(F_r, batch_r, mean_o, s2_o, sk_o, ku_o):
    Fv = F_r[...]
    bt = batch_r[...]
    P = Fv.shape[0]
    eps = jnp.float32(1e-4)

    gids = jax.lax.broadcasted_iota(jnp.int32, (P, G), 1)
    OH = (bt == gids).astype(jnp.float32)
    cnt1 = jnp.maximum(_graph_counts(OH), 1.0)

    sums = jax.lax.dot_general(OH, Fv, _DNT,
                               preferred_element_type=jnp.float32,
                               precision=_HP)
    mean = sums / cnt1
    cen = Fv - jax.lax.dot_general(OH, mean, _DN,
                                   preferred_element_type=jnp.float32,
                                   precision=_HP)
    cen = jnp.where(bt < G, cen, 0.0)
    c2 = cen * cen
    m2 = jax.lax.dot_general(OH, c2, _DNT,
                             preferred_element_type=jnp.float32,
                             precision=_HP) / cnt1
    m3 = jax.lax.dot_general(OH, c2 * cen, _DNT,
                             preferred_element_type=jnp.float32,
                             precision=_HP) / cnt1
    m4 = jax.lax.dot_general(OH, c2 * c2, _DNT,
                             preferred_element_type=jnp.float32,
                             precision=_HP) / cnt1

    mean_o[...] = mean
    s2_o[...] = jnp.sqrt(m2)
    sk_o[...] = m3 / jnp.maximum(m2 * jnp.sqrt(m2), eps)
    ku_o[...] = m4 / jnp.maximum(m2 * m2, eps)


def _median_body(F_r, batch_r, med_o, lo_s, hi_s, mid_s, cnt_s, cntg_s):
    t = pl.program_id(1)
    r = pl.program_id(2)
    nt = pl.num_programs(1)
    nr = pl.num_programs(2)

    topbit = jnp.uint32(0x80000000)
    allbits = jnp.uint32(0xFFFFFFFF)

    @pl.when((t == 0) & (r == 0))
    def _():
        lo_s[...] = jnp.zeros_like(lo_s)
        hi_s[...] = jnp.full_like(hi_s, allbits)
        cntg_s[...] = jnp.zeros_like(cntg_s)

    @pl.when(r == 0)
    def _():
        lo = lo_s[...]
        mid_s[...] = lo + ((hi_s[...] - lo) >> jnp.uint32(1))
        cnt_s[...] = jnp.zeros_like(cnt_s)

    Fv = F_r[...]
    bt = batch_r[...]
    BR = Fv.shape[0]
    gids = jax.lax.broadcasted_iota(jnp.int32, (BR, G), 1)
    OH = (bt == gids).astype(jnp.float32)

    @pl.when(t == 0)
    def _():
        cntg_s[...] += _graph_counts(OH)

    bits = jax.lax.bitcast_convert_type(Fv, jnp.uint32)
    u = jnp.where(bits >= topbit, bits ^ allbits, bits | topbit)

    mid = mid_s[...]
    mid_hi = (mid >> jnp.uint32(16)).astype(jnp.float32)
    mid_lo = (mid & jnp.uint32(0xFFFF)).astype(jnp.float32)
    thr_hi = jax.lax.dot_general(OH, mid_hi, _DN,
                                 preferred_element_type=jnp.float32,
                                 precision=_HP)
    thr_lo = jax.lax.dot_general(OH, mid_lo, _DN,
                                 preferred_element_type=jnp.float32,
                                 precision=_HP)
    thr = (thr_hi.astype(jnp.uint32) << jnp.uint32(16)) | \
        thr_lo.astype(jnp.uint32)
    le = (u <= thr).astype(jnp.float32)
    cnt_s[...] += jax.lax.dot_general(OH, le, _DNT,
                                      preferred_element_type=jnp.float32,
                                      precision=_HP)

    @pl.when(r == nr - 1)
    def _():
        cnt_i = cntg_s[...].astype(jnp.int32)
        need = (((cnt_i - 1) // 2) + 1).astype(jnp.float32)
        cond = cnt_s[...] >= need
        mid = mid_s[...]
        hi_s[...] = jnp.where(cond, mid, hi_s[...])
        lo_s[...] = jnp.where(cond, lo_s[...], mid + jnp.uint32(1))

        @pl.when(t == nt - 1)
        def _():
            med_u = hi_s[...]
            mbits = jnp.where(med_u >= topbit, med_u ^ topbit,
                              med_u ^ allbits)
            med_o[...] = jax.lax.bitcast_convert_type(mbits, jnp.float32)


def _stats(F, batch_col, P, CP):
    mean, s2, sk, ku = pl.pallas_call(
        _moments_body,
        grid=(CP // CB,),
        in_specs=[
            pl.BlockSpec((P, CB), lambda c: (0, c)),
            pl.BlockSpec((P, 1), lambda c: (0, 0)),
        ],
        out_specs=[pl.BlockSpec((G, CB), lambda c: (0, c))] * 4,
        out_shape=[jax.ShapeDtypeStruct((G, CP), jnp.float32)] * 4,
        compiler_params=pltpu.CompilerParams(
            dimension_semantics=("arbitrary",)),
    )(F, batch_col)
    BR = min(2048, P)
    med, = pl.pallas_call(
        _median_body,
        grid=(CP // CB, 32, P // BR),
        in_specs=[
            pl.BlockSpec((BR, CB), lambda c, t, r: (r, c)),
            pl.BlockSpec((BR, 1), lambda c, t, r: (r, 0)),
        ],
        out_specs=[pl.BlockSpec((G, CB), lambda c, t, r: (0, c))],
        out_shape=[jax.ShapeDtypeStruct((G, CP), jnp.float32)],
        scratch_shapes=[
            pltpu.VMEM((G, CB), jnp.uint32),
            pltpu.VMEM((G, CB), jnp.uint32),
            pltpu.VMEM((G, CB), jnp.uint32),
            pltpu.VMEM((G, CB), jnp.float32),
            pltpu.VMEM((G, 1), jnp.float32),
        ],
        compiler_params=pltpu.CompilerParams(
            dimension_semantics=("arbitrary", "arbitrary", "arbitrary")),
    )(F, batch_col)
    return mean, med, s2, sk, ku


_NC = 2
_NS = 16
_NW = _NC * _NS


def _zero_ref(ref, n):
    z = jnp.zeros((16,), jnp.float32)

    def bd(i, _):
        ref[pl.ds(i * 16, 16)] = z
        return 0

    lax.fori_loop(0, n // 16, bd, 0)


def _sc_deg(row3, P2, CHN):
    mesh = plsc.VectorSubcoreMesh(core_axis_name="c", subcore_axis_name="s")

    @functools.partial(
        pl.kernel, mesh=mesh,
        compiler_params=pltpu.CompilerParams(needs_layout_passes=False),
        out_type=jax.ShapeDtypeStruct((_NC, P2), jnp.float32),
        scratch_types=[
            pltpu.VMEM((CHN, 128), jnp.int32),
            pltpu.VMEM((128,), jnp.float32),
            pltpu.VMEM((P2,), jnp.float32),
            pltpu.VMEM_SHARED((P2,), jnp.float32),
        ],
    )
    def k(row_hbm, out_hbm, idx_v, ones_v, zbuf_v, shared):
        c = lax.axis_index("c")
        s = lax.axis_index("s")
        wid = s * _NC + c
        pltpu.sync_copy(row_hbm.at[wid], idx_v)
        def ob(i, _):
            ones_v[pl.ds(i * 16, 16)] = jnp.ones((16,), jnp.float32)
            return 0

        lax.fori_loop(0, 8, ob, 0)

        @pl.when(s == 0)
        def _():
            _zero_ref(zbuf_v, P2)
            pltpu.sync_copy(zbuf_v, shared)

        plsc.subcore_barrier()

        def bd(j, _):
            pltpu.sync_copy(ones_v, shared.at[idx_v.at[j]], add=True)
            return 0

        lax.fori_loop(0, CHN, bd, 0)
        plsc.subcore_barrier()

        @pl.when(s == 0)
        def _():
            pltpu.sync_copy(shared, out_hbm.at[c])

    return k(row3)


def _sc_edge_means(row3, col_flat, degp, f4, P2, CHN):
    mesh = plsc.VectorSubcoreMesh(core_axis_name="c", subcore_axis_name="s")

    @functools.partial(
        pl.kernel, mesh=mesh,
        compiler_params=pltpu.CompilerParams(needs_layout_passes=False),
        out_type=[
            jax.ShapeDtypeStruct((_NC, P2), jnp.float32),
            jax.ShapeDtypeStruct((_NC, P2), jnp.float32),
            jax.ShapeDtypeStruct((P2,), jnp.float32),
        ],
        scratch_types=[
            pltpu.VMEM((CHN, 128), jnp.int32),
            pltpu.VMEM((CHN * 128,), jnp.int32),
            pltpu.VMEM((P2,), jnp.float32),
            pltpu.VMEM((P2,), jnp.float32),
            pltpu.VMEM((P2,), jnp.float32),
            pltpu.VMEM((128,), jnp.float32),
            pltpu.VMEM_SHARED((P2,), jnp.float32),
            pltpu.VMEM_SHARED((P2,), jnp.float32),
        ],
    )
    def k(row_hbm, col_hbm, degp_hbm, f4_hbm, fe2_o, f3_o, f1_o,
          idx_v, col_v, dtab, tmp, f1tab, val_v, sh2, sh3):
        c = lax.axis_index("c")
        s = lax.axis_index("s")
        wid = s * _NC + c
        pltpu.sync_copy(row_hbm.at[wid], idx_v)
        pltpu.sync_copy(col_hbm.at[wid], col_v)
        pltpu.sync_copy(degp_hbm.at[0], dtab)
        pltpu.sync_copy(degp_hbm.at[1], tmp)

        def addbd(i, _):
            sl = pl.ds(i * 16, 16)
            dtab[sl] = dtab[sl] + tmp[sl]
            return 0

        lax.fori_loop(0, P2 // 16, addbd, 0)
        pltpu.sync_copy(f4_hbm, tmp)

        def f1bd(i, _):
            sl = pl.ds(i * 16, 16)
            d = dtab[sl]
            sm = d <= 1.0
            den = jnp.where(sm, 1.0, d * (d - 1.0))
            f1tab[sl] = jnp.where(sm, 0.0, (tmp[sl] - d) / den)
            return 0

        lax.fori_loop(0, P2 // 16, f1bd, 0)

        @pl.when((s == 0) & (c == 0))
        def _():
            pltpu.sync_copy(f1tab, f1_o)

        @pl.when(s == 0)
        def _():
            _zero_ref(tmp, P2)
            pltpu.sync_copy(tmp, sh2)
            pltpu.sync_copy(tmp, sh3)

        plsc.subcore_barrier()

        def chunk(j, _):
            def g16d(t, _c):
                ci = col_v[pl.ds(j * 128 + t * 16, 16)]
                val_v[pl.ds(t * 16, 16)] = plsc.load_gather(dtab, [ci])
                return 0

            def g16f(t, _c):
                ci = col_v[pl.ds(j * 128 + t * 16, 16)]
                val_v[pl.ds(t * 16, 16)] = plsc.load_gather(f1tab, [ci])
                return 0

            lax.fori_loop(0, 8, g16d, 0)
            pltpu.sync_copy(val_v, sh2.at[idx_v.at[j]], add=True)
            lax.fori_loop(0, 8, g16f, 0)
            pltpu.sync_copy(val_v, sh3.at[idx_v.at[j]], add=True)
            return 0

        lax.fori_loop(0, CHN, chunk, 0)
        plsc.subcore_barrier()

        @pl.when(s == 0)
        def _():
            pltpu.sync_copy(sh2, fe2_o.at[c])
            pltpu.sync_copy(sh3, f3_o.at[c])

    return k(row3, col_flat, degp, f4)


def kernel(x, edge_index, batch):
    N, DX = x.shape
    P = ((N + BM - 1) // BM) * BM
    P2 = P + 128
    row = edge_index[0].astype(jnp.int32)
    col = edge_index[1].astype(jnp.int32)
    E = row.shape[0]

    CHN = -(-E // (_NW * 128))
    EP = _NW * CHN * 128
    rowp = jnp.full((EP,), P2 - 1, jnp.int32).at[:E].set(row)
    colp = jnp.zeros((EP,), jnp.int32).at[:E].set(col)
    row3 = rowp.reshape(_NW, CHN, 128)
    col2 = colp.reshape(_NW, CHN * 128)

    degp = _sc_deg(row3, P2, CHN)
    deg_row = (degp[0:1, :P] + degp[1:2, :P])
    deg = deg_row[0, :N]

    Mf = jnp.zeros((P, P), jnp.float32).at[row, col].add(1.0)
    Mb = Mf.astype(jnp.bfloat16)
    idx = jnp.arange(N)
    Ab = (Mf > 0).astype(jnp.bfloat16).at[idx, idx].set(jnp.bfloat16(1))

    f4p, f5rp, f6rp = _ego(Ab, Mb, deg_row, P)
    f4 = f4p[:N, 0]
    f5r = f5rp[:N, 0]
    f6r = f6rp[:N, 0]

    f4c = jnp.zeros((P2,), jnp.float32).at[:P].set(f4p[:, 0])
    fe2n, f3n, f1p = _sc_edge_means(row3, col2, degp, f4c, P2, CHN)
    cmax = jnp.maximum(deg, 1.0)
    f1 = f1p[:N]
    feat2 = (fe2n[0, :N] + fe2n[1, :N]) / cmax
    f3 = (f3n[0, :N] + f3n[1, :N]) / cmax
    f5f = f5r - 2.0 * f4
    f6f = f6r - deg - 1.0

    topo = jnp.stack([deg, f1, feat2, f3, f4, f5f, f6f], axis=1)
    F = jnp.concatenate([x, topo], axis=1)
    C = DX + 7
    CP = ((C + CB - 1) // CB) * CB
    Fp = jnp.zeros((P, CP), jnp.float32).at[:N, :C].set(F)
    btp = jnp.full((P, 1), 255, jnp.int32).at[:N, 0].set(batch.astype(jnp.int32))

    mean, med, s2, sk, ku = _stats(Fp, btp, P, CP)
    return jnp.hstack([mean[:, :C], med[:, :C], s2[:, :C],
                       sk[:, :C], ku[:, :C]])


# speedup vs baseline: 166.9127x; 2.3084x over previous
import functools

import jax
import jax.numpy as jnp
from jax import lax
from jax.experimental import pallas as pl
from jax.experimental.pallas import tpu as pltpu
from jax.experimental.pallas import tpu_sc as plsc

G = 8
BM = 1024
BN = 2048
BK = 1024
CB = 128

_HP = jax.lax.Precision.HIGHEST
_DNT = (((0,), (0,)), ((), ()))
_DN = (((1,), (0,)), ((), ()))


def _ego_body(a_ik, m_kj, a_ij, deg_j, f4_o, f5_o, f6_o, acc):
    j = pl.program_id(1)
    k = pl.program_id(2)
    nk = pl.num_programs(2)

    @pl.when(k == 0)
    def _():
        acc[...] = jnp.zeros_like(acc)

    acc[...] += jax.lax.dot(a_ik[...], m_kj[...],
                            preferred_element_type=jnp.float32)

    @pl.when(k == nk - 1)
    def _():
        q = acc[...]
        aij = a_ij[...].astype(jnp.float32)
        f4p = jnp.sum(aij * q, axis=1, keepdims=True)
        b = ((aij + q) > 0.0).astype(jnp.float32)
        f5p = jnp.sum(b * deg_j[...], axis=1, keepdims=True)
        f6p = jnp.sum(b, axis=1, keepdims=True)

        @pl.when(j == 0)
        def _():
            f4_o[...] = f4p
            f5_o[...] = f5p
            f6_o[...] = f6p

        @pl.when(j != 0)
        def _():
            f4_o[...] += f4p
            f5_o[...] += f5p
            f6_o[...] += f6p


def _ego(A, M, deg_row, P):
    grid = (P // BM, P // BN, P // BK)
    out = pl.pallas_call(
        _ego_body,
        grid=grid,
        in_specs=[
            pl.BlockSpec((BM, BK), lambda i, j, k: (i, k)),
            pl.BlockSpec((BK, BN), lambda i, j, k: (k, j)),
            pl.BlockSpec((BM, BN), lambda i, j, k: (i, j)),
            pl.BlockSpec((1, BN), lambda i, j, k: (0, j)),
        ],
        out_specs=[
            pl.BlockSpec((BM, 1), lambda i, j, k: (i, 0)),
            pl.BlockSpec((BM, 1), lambda i, j, k: (i, 0)),
            pl.BlockSpec((BM, 1), lambda i, j, k: (i, 0)),
        ],
        out_shape=[jax.ShapeDtypeStruct((P, 1), jnp.float32)] * 3,
        scratch_shapes=[pltpu.VMEM((BM, BN), jnp.float32)],
        compiler_params=pltpu.CompilerParams(
            dimension_semantics=("parallel", "arbitrary", "arbitrary")),
    )(A, M, A, deg_row)
    return out


def _prep_body(n_nodes, m_r, mb_o, ab_o):
    r = pl.program_id(0)
    c = pl.program_id(1)
    m = m_r[...]
    BR, BC = m.shape
    mb_o[...] = m.astype(jnp.bfloat16)
    rid = r * BR + jax.lax.broadcasted_iota(jnp.int32, (BR, BC), 0)
    cid = c * BC + jax.lax.broadcasted_iota(jnp.int32, (BR, BC), 1)
    diag = (rid == cid) & (rid < n_nodes)
    ab = jnp.where(diag | (m > 0.0), 1.0, 0.0)
    ab_o[...] = ab.astype(jnp.bfloat16)


def _prep(Mf, N, P):
    import functools as _ft
    BR, BC = 512, 2048
    mb, ab = pl.pallas_call(
        _ft.partial(_prep_body, N),
        grid=(P // BR, P // BC),
        in_specs=[pl.BlockSpec((BR, BC), lambda r, c: (r, c))],
        out_specs=[pl.BlockSpec((BR, BC), lambda r, c: (r, c))] * 2,
        out_shape=[jax.ShapeDtypeStruct((P, P), jnp.bfloat16)] * 2,
        compiler_params=pltpu.CompilerParams(
            dimension_semantics=("parallel", "parallel")),
    )(Mf)
    return mb, ab


def _graph_counts(OH):
    ones_col = jnp.ones((OH.shape[0], 1), jnp.float32)
    return jax.lax.dot_general(OH, ones_col, _DNT,
                               preferred_element_type=jnp.float32,
                               precision=_HP)


def _moments_body(F_r, batch_r, mean_o, s2_o, sk_o, ku_o):
    Fv = F_r[...]
    bt = batch_r[...]
    P = Fv.shape[0]
    eps = jnp.float32(1e-4)

    gids = jax.lax.broadcasted_iota(jnp.int32, (P, G), 1)
    OH = (bt == gids).astype(jnp.float32)
    cnt1 = jnp.maximum(_graph_counts(OH), 1.0)

    sums = jax.lax.dot_general(OH, Fv, _DNT,
                               preferred_element_type=jnp.float32,
                               precision=_HP)
    mean = sums / cnt1
    cen = Fv - jax.lax.dot_general(OH, mean, _DN,
                                   preferred_element_type=jnp.float32,
                                   precision=_HP)
    cen = jnp.where(bt < G, cen, 0.0)
    c2 = cen * cen
    m2 = jax.lax.dot_general(OH, c2, _DNT,
                             preferred_element_type=jnp.float32,
                             precision=_HP) / cnt1
    m3 = jax.lax.dot_general(OH, c2 * cen, _DNT,
                             preferred_element_type=jnp.float32,
                             precision=_HP) / cnt1
    m4 = jax.lax.dot_general(OH, c2 * c2, _DNT,
                             preferred_element_type=jnp.float32,
                             precision=_HP) / cnt1

    mean_o[...] = mean
    s2_o[...] = jnp.sqrt(m2)
    sk_o[...] = m3 / jnp.maximum(m2 * jnp.sqrt(m2), eps)
    ku_o[...] = m4 / jnp.maximum(m2 * m2, eps)


def _median_body(F_r, batch_r, med_o, lo_s, hi_s, mid_s, cnt_s, cntg_s):
    t = pl.program_id(1)
    r = pl.program_id(2)
    nt = pl.num_programs(1)
    nr = pl.num_programs(2)

    topbit = jnp.uint32(0x80000000)
    allbits = jnp.uint32(0xFFFFFFFF)

    @pl.when((t == 0) & (r == 0))
    def _():
        lo_s[...] = jnp.zeros_like(lo_s)
        hi_s[...] = jnp.full_like(hi_s, allbits)
        cntg_s[...] = jnp.zeros_like(cntg_s)

    @pl.when(r == 0)
    def _():
        lo = lo_s[...]
        mid_s[...] = lo + ((hi_s[...] - lo) >> jnp.uint32(1))
        cnt_s[...] = jnp.zeros_like(cnt_s)

    Fv = F_r[...]
    bt = batch_r[...]
    BR = Fv.shape[0]
    gids = jax.lax.broadcasted_iota(jnp.int32, (BR, G), 1)
    OH = (bt == gids).astype(jnp.float32)

    @pl.when(t == 0)
    def _():
        cntg_s[...] += _graph_counts(OH)

    bits = jax.lax.bitcast_convert_type(Fv, jnp.uint32)
    u = jnp.where(bits >= topbit, bits ^ allbits, bits | topbit)

    mid = mid_s[...]
    mid_hi = (mid >> jnp.uint32(16)).astype(jnp.float32)
    mid_lo = (mid & jnp.uint32(0xFFFF)).astype(jnp.float32)
    thr_hi = jax.lax.dot_general(OH, mid_hi, _DN,
                                 preferred_element_type=jnp.float32,
                                 precision=_HP)
    thr_lo = jax.lax.dot_general(OH, mid_lo, _DN,
                                 preferred_element_type=jnp.float32,
                                 precision=_HP)
    thr = (thr_hi.astype(jnp.uint32) << jnp.uint32(16)) | \
        thr_lo.astype(jnp.uint32)
    le = (u <= thr).astype(jnp.float32)
    cnt_s[...] += jax.lax.dot_general(OH, le, _DNT,
                                      preferred_element_type=jnp.float32,
                                      precision=_HP)

    @pl.when(r == nr - 1)
    def _():
        cnt_i = cntg_s[...].astype(jnp.int32)
        need = (((cnt_i - 1) // 2) + 1).astype(jnp.float32)
        cond = cnt_s[...] >= need
        mid = mid_s[...]
        hi_s[...] = jnp.where(cond, mid, hi_s[...])
        lo_s[...] = jnp.where(cond, lo_s[...], mid + jnp.uint32(1))

        @pl.when(t == nt - 1)
        def _():
            med_u = hi_s[...]
            mbits = jnp.where(med_u >= topbit, med_u ^ topbit,
                              med_u ^ allbits)
            med_o[...] = jax.lax.bitcast_convert_type(mbits, jnp.float32)


def _stats(F, batch_col, P, CP):
    mean, s2, sk, ku = pl.pallas_call(
        _moments_body,
        grid=(CP // CB,),
        in_specs=[
            pl.BlockSpec((P, CB), lambda c: (0, c)),
            pl.BlockSpec((P, 1), lambda c: (0, 0)),
        ],
        out_specs=[pl.BlockSpec((G, CB), lambda c: (0, c))] * 4,
        out_shape=[jax.ShapeDtypeStruct((G, CP), jnp.float32)] * 4,
        compiler_params=pltpu.CompilerParams(
            dimension_semantics=("arbitrary",)),
    )(F, batch_col)
    BR = min(2048, P)
    med, = pl.pallas_call(
        _median_body,
        grid=(CP // CB, 32, P // BR),
        in_specs=[
            pl.BlockSpec((BR, CB), lambda c, t, r: (r, c)),
            pl.BlockSpec((BR, 1), lambda c, t, r: (r, 0)),
        ],
        out_specs=[pl.BlockSpec((G, CB), lambda c, t, r: (0, c))],
        out_shape=[jax.ShapeDtypeStruct((G, CP), jnp.float32)],
        scratch_shapes=[
            pltpu.VMEM((G, CB), jnp.uint32),
            pltpu.VMEM((G, CB), jnp.uint32),
            pltpu.VMEM((G, CB), jnp.uint32),
            pltpu.VMEM((G, CB), jnp.float32),
            pltpu.VMEM((G, 1), jnp.float32),
        ],
        compiler_params=pltpu.CompilerParams(
            dimension_semantics=("arbitrary", "arbitrary", "arbitrary")),
    )(F, batch_col)
    return mean, med, s2, sk, ku


_NC = 2
_NS = 16
_NW = _NC * _NS


def _zero_ref(ref, n):
    z = jnp.zeros((16,), jnp.float32)

    def bd(i, _):
        ref[pl.ds(i * 16, 16)] = z
        return 0

    lax.fori_loop(0, n // 16, bd, 0)


def _sc_deg(row3, P2, CHN):
    mesh = plsc.VectorSubcoreMesh(core_axis_name="c", subcore_axis_name="s")

    @functools.partial(
        pl.kernel, mesh=mesh,
        compiler_params=pltpu.CompilerParams(needs_layout_passes=False),
        out_type=jax.ShapeDtypeStruct((_NC, P2), jnp.float32),
        scratch_types=[
            pltpu.VMEM((CHN, 128), jnp.int32),
            pltpu.VMEM((128,), jnp.float32),
            pltpu.VMEM((P2,), jnp.float32),
            pltpu.VMEM_SHARED((P2,), jnp.float32),
        ],
    )
    def k(row_hbm, out_hbm, idx_v, ones_v, zbuf_v, shared):
        c = lax.axis_index("c")
        s = lax.axis_index("s")
        wid = s * _NC + c
        pltpu.sync_copy(row_hbm.at[wid], idx_v)
        def ob(i, _):
            ones_v[pl.ds(i * 16, 16)] = jnp.ones((16,), jnp.float32)
            return 0

        lax.fori_loop(0, 8, ob, 0)

        @pl.when(s == 0)
        def _():
            _zero_ref(zbuf_v, P2)
            pltpu.sync_copy(zbuf_v, shared)

        plsc.subcore_barrier()

        def bd(j, _):
            pltpu.sync_copy(ones_v, shared.at[idx_v.at[j]], add=True)
            return 0

        lax.fori_loop(0, CHN, bd, 0)
        plsc.subcore_barrier()

        @pl.when(s == 0)
        def _():
            pltpu.sync_copy(shared, out_hbm.at[c])

    return k(row3)


def _sc_edge_means(row3, col_flat, degp, f4, P2, CHN):
    mesh = plsc.VectorSubcoreMesh(core_axis_name="c", subcore_axis_name="s")

    @functools.partial(
        pl.kernel, mesh=mesh,
        compiler_params=pltpu.CompilerParams(needs_layout_passes=False),
        out_type=[
            jax.ShapeDtypeStruct((_NC, P2), jnp.float32),
            jax.ShapeDtypeStruct((_NC, P2), jnp.float32),
            jax.ShapeDtypeStruct((P2,), jnp.float32),
        ],
        scratch_types=[
            pltpu.VMEM((CHN, 128), jnp.int32),
            pltpu.VMEM((CHN * 128,), jnp.int32),
            pltpu.VMEM((P2,), jnp.float32),
            pltpu.VMEM((P2,), jnp.float32),
            pltpu.VMEM((P2,), jnp.float32),
            pltpu.VMEM((128,), jnp.float32),
            pltpu.VMEM_SHARED((P2,), jnp.float32),
            pltpu.VMEM_SHARED((P2,), jnp.float32),
        ],
    )
    def k(row_hbm, col_hbm, degp_hbm, f4_hbm, fe2_o, f3_o, f1_o,
          idx_v, col_v, dtab, tmp, f1tab, val_v, sh2, sh3):
        c = lax.axis_index("c")
        s = lax.axis_index("s")
        wid = s * _NC + c
        pltpu.sync_copy(row_hbm.at[wid], idx_v)
        pltpu.sync_copy(col_hbm.at[wid], col_v)
        pltpu.sync_copy(degp_hbm.at[0], dtab)
        pltpu.sync_copy(degp_hbm.at[1], tmp)

        def addbd(i, _):
            sl = pl.ds(i * 16, 16)
            dtab[sl] = dtab[sl] + tmp[sl]
            return 0

        lax.fori_loop(0, P2 // 16, addbd, 0)
        pltpu.sync_copy(f4_hbm, tmp)

        def f1bd(i, _):
            sl = pl.ds(i * 16, 16)
            d = dtab[sl]
            sm = d <= 1.0
            den = jnp.where(sm, 1.0, d * (d - 1.0))
            f1tab[sl] = jnp.where(sm, 0.0, (tmp[sl] - d) / den)
            return 0

        lax.fori_loop(0, P2 // 16, f1bd, 0)

        @pl.when((s == 0) & (c == 0))
        def _():
            pltpu.sync_copy(f1tab, f1_o)

        @pl.when(s == 0)
        def _():
            _zero_ref(tmp, P2)
            pltpu.sync_copy(tmp, sh2)
            pltpu.sync_copy(tmp, sh3)

        plsc.subcore_barrier()

        def chunk(j, _):
            def g16d(t, _c):
                ci = col_v[pl.ds(j * 128 + t * 16, 16)]
                val_v[pl.ds(t * 16, 16)] = plsc.load_gather(dtab, [ci])
                return 0

            def g16f(t, _c):
                ci = col_v[pl.ds(j * 128 + t * 16, 16)]
                val_v[pl.ds(t * 16, 16)] = plsc.load_gather(f1tab, [ci])
                return 0

            lax.fori_loop(0, 8, g16d, 0)
            pltpu.sync_copy(val_v, sh2.at[idx_v.at[j]], add=True)
            lax.fori_loop(0, 8, g16f, 0)
            pltpu.sync_copy(val_v, sh3.at[idx_v.at[j]], add=True)
            return 0

        lax.fori_loop(0, CHN, chunk, 0)
        plsc.subcore_barrier()

        @pl.when(s == 0)
        def _():
            pltpu.sync_copy(sh2, fe2_o.at[c])
            pltpu.sync_copy(sh3, f3_o.at[c])

    return k(row3, col_flat, degp, f4)


def kernel(x, edge_index, batch):
    N, DX = x.shape
    P = ((N + BM - 1) // BM) * BM
    P2 = P + 128
    row = edge_index[0].astype(jnp.int32)
    col = edge_index[1].astype(jnp.int32)
    E = row.shape[0]

    CHN = -(-E // (_NW * 128))
    EP = _NW * CHN * 128
    rowp = jnp.full((EP,), P2 - 1, jnp.int32).at[:E].set(row)
    colp = jnp.zeros((EP,), jnp.int32).at[:E].set(col)
    row3 = rowp.reshape(_NW, CHN, 128)
    col2 = colp.reshape(_NW, CHN * 128)

    degp = _sc_deg(row3, P2, CHN)
    deg_row = (degp[0:1, :P] + degp[1:2, :P])
    deg = deg_row[0, :N]

    Mf = jnp.zeros((P, P), jnp.float32).at[row, col].add(1.0)
    Mb, Ab = _prep(Mf, N, P)

    f4p, f5rp, f6rp = _ego(Ab, Mb, deg_row, P)
    f4 = f4p[:N, 0]
    f5r = f5rp[:N, 0]
    f6r = f6rp[:N, 0]

    f4c = jnp.zeros((P2,), jnp.float32).at[:P].set(f4p[:, 0])
    fe2n, f3n, f1p = _sc_edge_means(row3, col2, degp, f4c, P2, CHN)
    cmax = jnp.maximum(deg, 1.0)
    f1 = f1p[:N]
    feat2 = (fe2n[0, :N] + fe2n[1, :N]) / cmax
    f3 = (f3n[0, :N] + f3n[1, :N]) / cmax
    f5f = f5r - 2.0 * f4
    f6f = f6r - deg - 1.0

    topo = jnp.stack([deg, f1, feat2, f3, f4, f5f, f6f], axis=1)
    F = jnp.concatenate([x, topo], axis=1)
    C = DX + 7
    CP = ((C + CB - 1) // CB) * CB
    Fp = jnp.zeros((P, CP), jnp.float32).at[:N, :C].set(F)
    btp = jnp.full((P, 1), 255, jnp.int32).at[:N, 0].set(batch.astype(jnp.int32))

    mean, med, s2, sk, ku = _stats(Fp, btp, P, CP)
    return jnp.hstack([mean[:, :C], med[:, :C], s2[:, :C],
                       sk[:, :C], ku[:, :C]])


# speedup vs baseline: 167.9131x; 1.0060x over previous
import functools

import jax
import jax.numpy as jnp
from jax import lax
from jax.experimental import pallas as pl
from jax.experimental.pallas import tpu as pltpu
from jax.experimental.pallas import tpu_sc as plsc

G = 8
BM = 1024
BN = 2048
BK = 1024
CB = 128

_HP = jax.lax.Precision.HIGHEST
_DNT = (((0,), (0,)), ((), ()))
_DN = (((1,), (0,)), ((), ()))


def _ego_body(a_ik, m_kj, a_ij, deg_j, f4_o, f5_o, f6_o, acc):
    j = pl.program_id(1)
    k = pl.program_id(2)
    nk = pl.num_programs(2)

    @pl.when(k == 0)
    def _():
        acc[...] = jnp.zeros_like(acc)

    acc[...] += jax.lax.dot(a_ik[...], m_kj[...],
                            preferred_element_type=jnp.int32)

    @pl.when(k == nk - 1)
    def _():
        q = acc[...]
        aij = a_ij[...].astype(jnp.int32)
        f4p = jnp.sum((aij * q).astype(jnp.float32), axis=1, keepdims=True)
        b = ((aij + q) > 0).astype(jnp.float32)
        f5p = jnp.sum(b * deg_j[...], axis=1, keepdims=True)
        f6p = jnp.sum(b, axis=1, keepdims=True)

        @pl.when(j == 0)
        def _():
            f4_o[...] = f4p
            f5_o[...] = f5p
            f6_o[...] = f6p

        @pl.when(j != 0)
        def _():
            f4_o[...] += f4p
            f5_o[...] += f5p
            f6_o[...] += f6p


def _ego(A, M, deg_row, P):
    grid = (P // BM, P // BN, P // BK)
    out = pl.pallas_call(
        _ego_body,
        grid=grid,
        in_specs=[
            pl.BlockSpec((BM, BK), lambda i, j, k: (i, k)),
            pl.BlockSpec((BK, BN), lambda i, j, k: (k, j)),
            pl.BlockSpec((BM, BN), lambda i, j, k: (i, j)),
            pl.BlockSpec((1, BN), lambda i, j, k: (0, j)),
        ],
        out_specs=[
            pl.BlockSpec((BM, 1), lambda i, j, k: (i, 0)),
            pl.BlockSpec((BM, 1), lambda i, j, k: (i, 0)),
            pl.BlockSpec((BM, 1), lambda i, j, k: (i, 0)),
        ],
        out_shape=[jax.ShapeDtypeStruct((P, 1), jnp.float32)] * 3,
        scratch_shapes=[pltpu.VMEM((BM, BN), jnp.int32)],
        compiler_params=pltpu.CompilerParams(
            dimension_semantics=("parallel", "arbitrary", "arbitrary")),
    )(A, M, A, deg_row)
    return out


def _prep_body(n_nodes, m_r, mb_o, ab_o):
    r = pl.program_id(0)
    c = pl.program_id(1)
    m = m_r[...]
    BR, BC = m.shape
    mb_o[...] = m.astype(jnp.int8)
    rid = r * BR + jax.lax.broadcasted_iota(jnp.int32, (BR, BC), 0)
    cid = c * BC + jax.lax.broadcasted_iota(jnp.int32, (BR, BC), 1)
    diag = (rid == cid) & (rid < n_nodes)
    ab = jnp.where(diag | (m > 0.0), 1, 0)
    ab_o[...] = ab.astype(jnp.int8)


def _prep(Mf, N, P):
    import functools as _ft
    BR, BC = 512, 2048
    mb, ab = pl.pallas_call(
        _ft.partial(_prep_body, N),
        grid=(P // BR, P // BC),
        in_specs=[pl.BlockSpec((BR, BC), lambda r, c: (r, c))],
        out_specs=[pl.BlockSpec((BR, BC), lambda r, c: (r, c))] * 2,
        out_shape=[jax.ShapeDtypeStruct((P, P), jnp.int8)] * 2,
        compiler_params=pltpu.CompilerParams(
            dimension_semantics=("parallel", "parallel")),
    )(Mf)
    return mb, ab


def _graph_counts(OH):
    ones_col = jnp.ones((OH.shape[0], 1), jnp.float32)
    return jax.lax.dot_general(OH, ones_col, _DNT,
                               preferred_element_type=jnp.float32,
                               precision=_HP)


def _moments_body(F_r, batch_r, mean_o, s2_o, sk_o, ku_o):
    Fv = F_r[...]
    bt = batch_r[...]
    P = Fv.shape[0]
    eps = jnp.float32(1e-4)

    gids = jax.lax.broadcasted_iota(jnp.int32, (P, G), 1)
    OH = (bt == gids).astype(jnp.float32)
    cnt1 = jnp.maximum(_graph_counts(OH), 1.0)

    sums = jax.lax.dot_general(OH, Fv, _DNT,
                               preferred_element_type=jnp.float32,
                               precision=_HP)
    mean = sums / cnt1
    cen = Fv - jax.lax.dot_general(OH, mean, _DN,
                                   preferred_element_type=jnp.float32,
                                   precision=_HP)
    cen = jnp.where(bt < G, cen, 0.0)
    c2 = cen * cen
    m2 = jax.lax.dot_general(OH, c2, _DNT,
                             preferred_element_type=jnp.float32,
                             precision=_HP) / cnt1
    m3 = jax.lax.dot_general(OH, c2 * cen, _DNT,
                             preferred_element_type=jnp.float32,
                             precision=_HP) / cnt1
    m4 = jax.lax.dot_general(OH, c2 * c2, _DNT,
                             preferred_element_type=jnp.float32,
                             precision=_HP) / cnt1

    mean_o[...] = mean
    s2_o[...] = jnp.sqrt(m2)
    sk_o[...] = m3 / jnp.maximum(m2 * jnp.sqrt(m2), eps)
    ku_o[...] = m4 / jnp.maximum(m2 * m2, eps)


def _median_body(F_r, batch_r, med_o, lo_s, hi_s, mid_s, cnt_s, cntg_s):
    t = pl.program_id(1)
    r = pl.program_id(2)
    nt = pl.num_programs(1)
    nr = pl.num_programs(2)

    topbit = jnp.uint32(0x80000000)
    allbits = jnp.uint32(0xFFFFFFFF)

    @pl.when((t == 0) & (r == 0))
    def _():
        lo_s[...] = jnp.zeros_like(lo_s)
        hi_s[...] = jnp.full_like(hi_s, allbits)
        cntg_s[...] = jnp.zeros_like(cntg_s)

    @pl.when(r == 0)
    def _():
        lo = lo_s[...]
        mid_s[...] = lo + ((hi_s[...] - lo) >> jnp.uint32(1))
        cnt_s[...] = jnp.zeros_like(cnt_s)

    Fv = F_r[...]
    bt = batch_r[...]
    BR = Fv.shape[0]
    gids = jax.lax.broadcasted_iota(jnp.int32, (BR, G), 1)
    OH = (bt == gids).astype(jnp.float32)

    @pl.when(t == 0)
    def _():
        cntg_s[...] += _graph_counts(OH)

    bits = jax.lax.bitcast_convert_type(Fv, jnp.uint32)
    u = jnp.where(bits >= topbit, bits ^ allbits, bits | topbit)

    mid = mid_s[...]
    mid_hi = (mid >> jnp.uint32(16)).astype(jnp.float32)
    mid_lo = (mid & jnp.uint32(0xFFFF)).astype(jnp.float32)
    thr_hi = jax.lax.dot_general(OH, mid_hi, _DN,
                                 preferred_element_type=jnp.float32,
                                 precision=_HP)
    thr_lo = jax.lax.dot_general(OH, mid_lo, _DN,
                                 preferred_element_type=jnp.float32,
                                 precision=_HP)
    thr = (thr_hi.astype(jnp.uint32) << jnp.uint32(16)) | \
        thr_lo.astype(jnp.uint32)
    le = (u <= thr).astype(jnp.float32)
    cnt_s[...] += jax.lax.dot_general(OH, le, _DNT,
                                      preferred_element_type=jnp.float32,
                                      precision=_HP)

    @pl.when(r == nr - 1)
    def _():
        cnt_i = cntg_s[...].astype(jnp.int32)
        need = (((cnt_i - 1) // 2) + 1).astype(jnp.float32)
        cond = cnt_s[...] >= need
        mid = mid_s[...]
        hi_s[...] = jnp.where(cond, mid, hi_s[...])
        lo_s[...] = jnp.where(cond, lo_s[...], mid + jnp.uint32(1))

        @pl.when(t == nt - 1)
        def _():
            med_u = hi_s[...]
            mbits = jnp.where(med_u >= topbit, med_u ^ topbit,
                              med_u ^ allbits)
            med_o[...] = jax.lax.bitcast_convert_type(mbits, jnp.float32)


def _stats(F, batch_col, P, CP):
    mean, s2, sk, ku = pl.pallas_call(
        _moments_body,
        grid=(CP // CB,),
        in_specs=[
            pl.BlockSpec((P, CB), lambda c: (0, c)),
            pl.BlockSpec((P, 1), lambda c: (0, 0)),
        ],
        out_specs=[pl.BlockSpec((G, CB), lambda c: (0, c))] * 4,
        out_shape=[jax.ShapeDtypeStruct((G, CP), jnp.float32)] * 4,
        compiler_params=pltpu.CompilerParams(
            dimension_semantics=("arbitrary",)),
    )(F, batch_col)
    BR = min(2048, P)
    med, = pl.pallas_call(
        _median_body,
        grid=(CP // CB, 32, P // BR),
        in_specs=[
            pl.BlockSpec((BR, CB), lambda c, t, r: (r, c)),
            pl.BlockSpec((BR, 1), lambda c, t, r: (r, 0)),
        ],
        out_specs=[pl.BlockSpec((G, CB), lambda c, t, r: (0, c))],
        out_shape=[jax.ShapeDtypeStruct((G, CP), jnp.float32)],
        scratch_shapes=[
            pltpu.VMEM((G, CB), jnp.uint32),
            pltpu.VMEM((G, CB), jnp.uint32),
            pltpu.VMEM((G, CB), jnp.uint32),
            pltpu.VMEM((G, CB), jnp.float32),
            pltpu.VMEM((G, 1), jnp.float32),
        ],
        compiler_params=pltpu.CompilerParams(
            dimension_semantics=("arbitrary", "arbitrary", "arbitrary")),
    )(F, batch_col)
    return mean, med, s2, sk, ku


_NC = 2
_NS = 16
_NW = _NC * _NS


def _zero_ref(ref, n):
    z = jnp.zeros((16,), jnp.float32)

    def bd(i, _):
        ref[pl.ds(i * 16, 16)] = z
        return 0

    lax.fori_loop(0, n // 16, bd, 0)


def _sc_deg(row3, P2, CHN):
    mesh = plsc.VectorSubcoreMesh(core_axis_name="c", subcore_axis_name="s")

    @functools.partial(
        pl.kernel, mesh=mesh,
        compiler_params=pltpu.CompilerParams(needs_layout_passes=False),
        out_type=jax.ShapeDtypeStruct((_NC, P2), jnp.float32),
        scratch_types=[
            pltpu.VMEM((CHN, 128), jnp.int32),
            pltpu.VMEM((128,), jnp.float32),
            pltpu.VMEM((P2,), jnp.float32),
            pltpu.VMEM_SHARED((P2,), jnp.float32),
        ],
    )
    def k(row_hbm, out_hbm, idx_v, ones_v, zbuf_v, shared):
        c = lax.axis_index("c")
        s = lax.axis_index("s")
        wid = s * _NC + c
        pltpu.sync_copy(row_hbm.at[wid], idx_v)
        def ob(i, _):
            ones_v[pl.ds(i * 16, 16)] = jnp.ones((16,), jnp.float32)
            return 0

        lax.fori_loop(0, 8, ob, 0)

        @pl.when(s == 0)
        def _():
            _zero_ref(zbuf_v, P2)
            pltpu.sync_copy(zbuf_v, shared)

        plsc.subcore_barrier()

        def bd(j, _):
            pltpu.sync_copy(ones_v, shared.at[idx_v.at[j]], add=True)
            return 0

        lax.fori_loop(0, CHN, bd, 0)
        plsc.subcore_barrier()

        @pl.when(s == 0)
        def _():
            pltpu.sync_copy(shared, out_hbm.at[c])

    return k(row3)


def _sc_edge_means(row3, col_flat, degp, f4, P2, CHN):
    mesh = plsc.VectorSubcoreMesh(core_axis_name="c", subcore_axis_name="s")

    @functools.partial(
        pl.kernel, mesh=mesh,
        compiler_params=pltpu.CompilerParams(needs_layout_passes=False),
        out_type=[
            jax.ShapeDtypeStruct((_NC, P2), jnp.float32),
            jax.ShapeDtypeStruct((_NC, P2), jnp.float32),
            jax.ShapeDtypeStruct((P2,), jnp.float32),
        ],
        scratch_types=[
            pltpu.VMEM((CHN, 128), jnp.int32),
            pltpu.VMEM((CHN * 128,), jnp.int32),
            pltpu.VMEM((P2,), jnp.float32),
            pltpu.VMEM((P2,), jnp.float32),
            pltpu.VMEM((P2,), jnp.float32),
            pltpu.VMEM((128,), jnp.float32),
            pltpu.VMEM_SHARED((P2,), jnp.float32),
            pltpu.VMEM_SHARED((P2,), jnp.float32),
        ],
    )
    def k(row_hbm, col_hbm, degp_hbm, f4_hbm, fe2_o, f3_o, f1_o,
          idx_v, col_v, dtab, tmp, f1tab, val_v, sh2, sh3):
        c = lax.axis_index("c")
        s = lax.axis_index("s")
        wid = s * _NC + c
        pltpu.sync_copy(row_hbm.at[wid], idx_v)
        pltpu.sync_copy(col_hbm.at[wid], col_v)
        pltpu.sync_copy(degp_hbm.at[0], dtab)
        pltpu.sync_copy(degp_hbm.at[1], tmp)

        def addbd(i, _):
            sl = pl.ds(i * 16, 16)
            dtab[sl] = dtab[sl] + tmp[sl]
            return 0

        lax.fori_loop(0, P2 // 16, addbd, 0)
        pltpu.sync_copy(f4_hbm, tmp)

        def f1bd(i, _):
            sl = pl.ds(i * 16, 16)
            d = dtab[sl]
            sm = d <= 1.0
            den = jnp.where(sm, 1.0, d * (d - 1.0))
            f1tab[sl] = jnp.where(sm, 0.0, (tmp[sl] - d) / den)
            return 0

        lax.fori_loop(0, P2 // 16, f1bd, 0)

        @pl.when((s == 0) & (c == 0))
        def _():
            pltpu.sync_copy(f1tab, f1_o)

        @pl.when(s == 0)
        def _():
            _zero_ref(tmp, P2)
            pltpu.sync_copy(tmp, sh2)
            pltpu.sync_copy(tmp, sh3)

        plsc.subcore_barrier()

        def chunk(j, _):
            def g16d(t, _c):
                ci = col_v[pl.ds(j * 128 + t * 16, 16)]
                val_v[pl.ds(t * 16, 16)] = plsc.load_gather(dtab, [ci])
                return 0

            def g16f(t, _c):
                ci = col_v[pl.ds(j * 128 + t * 16, 16)]
                val_v[pl.ds(t * 16, 16)] = plsc.load_gather(f1tab, [ci])
                return 0

            lax.fori_loop(0, 8, g16d, 0)
            pltpu.sync_copy(val_v, sh2.at[idx_v.at[j]], add=True)
            lax.fori_loop(0, 8, g16f, 0)
            pltpu.sync_copy(val_v, sh3.at[idx_v.at[j]], add=True)
            return 0

        lax.fori_loop(0, CHN, chunk, 0)
        plsc.subcore_barrier()

        @pl.when(s == 0)
        def _():
            pltpu.sync_copy(sh2, fe2_o.at[c])
            pltpu.sync_copy(sh3, f3_o.at[c])

    return k(row3, col_flat, degp, f4)


def kernel(x, edge_index, batch):
    N, DX = x.shape
    P = ((N + BM - 1) // BM) * BM
    P2 = P + 128
    row = edge_index[0].astype(jnp.int32)
    col = edge_index[1].astype(jnp.int32)
    E = row.shape[0]

    CHN = -(-E // (_NW * 128))
    EP = _NW * CHN * 128
    rowp = jnp.full((EP,), P2 - 1, jnp.int32).at[:E].set(row)
    colp = jnp.zeros((EP,), jnp.int32).at[:E].set(col)
    row3 = rowp.reshape(_NW, CHN, 128)
    col2 = colp.reshape(_NW, CHN * 128)

    degp = _sc_deg(row3, P2, CHN)
    deg_row = (degp[0:1, :P] + degp[1:2, :P])
    deg = deg_row[0, :N]

    Mf = jnp.zeros((P, P), jnp.float32).at[row, col].add(1.0)
    Mb, Ab = _prep(Mf, N, P)

    f4p, f5rp, f6rp = _ego(Ab, Mb, deg_row, P)
    f4 = f4p[:N, 0]
    f5r = f5rp[:N, 0]
    f6r = f6rp[:N, 0]

    f4c = jnp.zeros((P2,), jnp.float32).at[:P].set(f4p[:, 0])
    fe2n, f3n, f1p = _sc_edge_means(row3, col2, degp, f4c, P2, CHN)
    cmax = jnp.maximum(deg, 1.0)
    f1 = f1p[:N]
    feat2 = (fe2n[0, :N] + fe2n[1, :N]) / cmax
    f3 = (f3n[0, :N] + f3n[1, :N]) / cmax
    f5f = f5r - 2.0 * f4
    f6f = f6r - deg - 1.0

    topo = jnp.stack([deg, f1, feat2, f3, f4, f5f, f6f], axis=1)
    F = jnp.concatenate([x, topo], axis=1)
    C = DX + 7
    CP = ((C + CB - 1) // CB) * CB
    Fp = jnp.zeros((P, CP), jnp.float32).at[:N, :C].set(F)
    btp = jnp.full((P, 1), 255, jnp.int32).at[:N, 0].set(batch.astype(jnp.int32))

    mean, med, s2, sk, ku = _stats(Fp, btp, P, CP)
    return jnp.hstack([mean[:, :C], med[:, :C], s2[:, :C],
                       sk[:, :C], ku[:, :C]])


# speedup vs baseline: 182.9319x; 1.0894x over previous
import functools

import jax
import jax.numpy as jnp
from jax import lax
from jax.experimental import pallas as pl
from jax.experimental.pallas import tpu as pltpu
from jax.experimental.pallas import tpu_sc as plsc

G = 8
BM = 1024
BN = 2048
BK = 1024
CB = 128

_HP = jax.lax.Precision.HIGHEST
_DNT = (((0,), (0,)), ((), ()))
_DN = (((1,), (0,)), ((), ()))


def _ego_body(a_ik, m_kj, a_ij, deg_j, f4_o, f5_o, f6_o, acc):
    j = pl.program_id(1)
    k = pl.program_id(2)
    nk = pl.num_programs(2)

    @pl.when(k == 0)
    def _():
        acc[...] = jnp.zeros_like(acc)

    acc[...] += jax.lax.dot(a_ik[...], m_kj[...],
                            preferred_element_type=jnp.int32)

    @pl.when(k == nk - 1)
    def _():
        q = acc[...]
        aij = a_ij[...].astype(jnp.int32)
        f4p = jnp.sum((aij * q).astype(jnp.float32), axis=1, keepdims=True)
        b = ((aij + q) > 0).astype(jnp.float32)
        f5p = jnp.sum(b * deg_j[...], axis=1, keepdims=True)
        f6p = jnp.sum(b, axis=1, keepdims=True)

        @pl.when(j == 0)
        def _():
            f4_o[...] = f4p
            f5_o[...] = f5p
            f6_o[...] = f6p

        @pl.when(j != 0)
        def _():
            f4_o[...] += f4p
            f5_o[...] += f5p
            f6_o[...] += f6p


def _ego(A, M, deg_row, P):
    grid = (P // BM, P // BN, P // BK)
    out = pl.pallas_call(
        _ego_body,
        grid=grid,
        in_specs=[
            pl.BlockSpec((BM, BK), lambda i, j, k: (i, k)),
            pl.BlockSpec((BK, BN), lambda i, j, k: (k, j)),
            pl.BlockSpec((BM, BN), lambda i, j, k: (i, j)),
            pl.BlockSpec((1, BN), lambda i, j, k: (0, j)),
        ],
        out_specs=[
            pl.BlockSpec((BM, 1), lambda i, j, k: (i, 0)),
            pl.BlockSpec((BM, 1), lambda i, j, k: (i, 0)),
            pl.BlockSpec((BM, 1), lambda i, j, k: (i, 0)),
        ],
        out_shape=[jax.ShapeDtypeStruct((P, 1), jnp.float32)] * 3,
        scratch_shapes=[pltpu.VMEM((BM, BN), jnp.int32)],
        compiler_params=pltpu.CompilerParams(
            dimension_semantics=("parallel", "arbitrary", "arbitrary")),
    )(A, M, A, deg_row)
    return out


def _prep_body(n_nodes, m_r, mb_o, ab_o):
    r = pl.program_id(0)
    c = pl.program_id(1)
    m = m_r[...]
    BR, BC = m.shape
    mb_o[...] = m.astype(jnp.int8)
    rid = r * BR + jax.lax.broadcasted_iota(jnp.int32, (BR, BC), 0)
    cid = c * BC + jax.lax.broadcasted_iota(jnp.int32, (BR, BC), 1)
    diag = (rid == cid) & (rid < n_nodes)
    ab = jnp.where(diag | (m > 0.0), 1, 0)
    ab_o[...] = ab.astype(jnp.int8)


def _prep(Mf, N, P):
    import functools as _ft
    BR, BC = 512, 2048
    mb, ab = pl.pallas_call(
        _ft.partial(_prep_body, N),
        grid=(P // BR, P // BC),
        in_specs=[pl.BlockSpec((BR, BC), lambda r, c: (r, c))],
        out_specs=[pl.BlockSpec((BR, BC), lambda r, c: (r, c))] * 2,
        out_shape=[jax.ShapeDtypeStruct((P, P), jnp.int8)] * 2,
        compiler_params=pltpu.CompilerParams(
            dimension_semantics=("parallel", "parallel")),
    )(Mf)
    return mb, ab


def _graph_counts(OH):
    ones_col = jnp.ones((OH.shape[0], 1), jnp.float32)
    return jax.lax.dot_general(OH, ones_col, _DNT,
                               preferred_element_type=jnp.float32,
                               precision=_HP)


def _moments_body(F_r, batch_r, mean_o, s2_o, sk_o, ku_o):
    Fv = F_r[...]
    bt = batch_r[...]
    P = Fv.shape[0]
    eps = jnp.float32(1e-4)

    gids = jax.lax.broadcasted_iota(jnp.int32, (P, G), 1)
    OH = (bt == gids).astype(jnp.float32)
    cnt1 = jnp.maximum(_graph_counts(OH), 1.0)

    sums = jax.lax.dot_general(OH, Fv, _DNT,
                               preferred_element_type=jnp.float32,
                               precision=_HP)
    mean = sums / cnt1
    cen = Fv - jax.lax.dot_general(OH, mean, _DN,
                                   preferred_element_type=jnp.float32,
                                   precision=_HP)
    cen = jnp.where(bt < G, cen, 0.0)
    c2 = cen * cen
    m2 = jax.lax.dot_general(OH, c2, _DNT,
                             preferred_element_type=jnp.float32,
                             precision=_HP) / cnt1
    m3 = jax.lax.dot_general(OH, c2 * cen, _DNT,
                             preferred_element_type=jnp.float32,
                             precision=_HP) / cnt1
    m4 = jax.lax.dot_general(OH, c2 * c2, _DNT,
                             preferred_element_type=jnp.float32,
                             precision=_HP) / cnt1

    mean_o[...] = mean
    s2_o[...] = jnp.sqrt(m2)
    sk_o[...] = m3 / jnp.maximum(m2 * jnp.sqrt(m2), eps)
    ku_o[...] = m4 / jnp.maximum(m2 * m2, eps)


def _median_body(F_r, batch_r, med_o, lo_s, hi_s, cntg_s):
    t = pl.program_id(1)
    nt = pl.num_programs(1)

    topbit = jnp.uint32(0x80000000)
    allbits = jnp.uint32(0xFFFFFFFF)
    Fv = F_r[...]
    bt = batch_r[...]
    P = Fv.shape[0]
    gids = jax.lax.broadcasted_iota(jnp.int32, (P, G), 1)
    OH = (bt == gids).astype(jnp.float32)

    @pl.when(t == 0)
    def _():
        lo_s[...] = jnp.zeros_like(lo_s)
        hi_s[...] = jnp.full_like(hi_s, allbits)
        ones_col = jnp.ones((P, 1), jnp.float32)
        cntg_s[...] = jax.lax.dot_general(
            OH, ones_col, _DNT, preferred_element_type=jnp.float32)

    bits = jax.lax.bitcast_convert_type(Fv, jnp.uint32)
    u = jnp.where(bits >= topbit, bits ^ allbits, bits | topbit)

    lo = lo_s[...]
    hi = hi_s[...]
    mid = lo + ((hi - lo) >> jnp.uint32(1))
    m255 = jnp.uint32(0xFF)
    thr = jnp.zeros_like(u)
    for sh in (0, 8, 16, 24):
        piece = ((mid >> jnp.uint32(sh)) & m255).astype(jnp.float32)
        pb = jax.lax.dot_general(OH, piece, _DN,
                                 preferred_element_type=jnp.float32)
        thr = thr | (pb.astype(jnp.uint32) << jnp.uint32(sh))
    le = (u <= thr).astype(jnp.float32)
    cle = jax.lax.dot_general(OH, le, _DNT,
                              preferred_element_type=jnp.float32)
    cnt_i = cntg_s[...].astype(jnp.int32)
    need = (((cnt_i - 1) // 2) + 1).astype(jnp.float32)
    cond = cle >= need
    hi = jnp.where(cond, mid, hi)
    lo = jnp.where(cond, lo, mid + jnp.uint32(1))
    hi_s[...] = hi
    lo_s[...] = lo

    @pl.when(t == nt - 1)
    def _():
        mbits = jnp.where(hi >= topbit, hi ^ topbit, hi ^ allbits)
        med_o[...] = jax.lax.bitcast_convert_type(mbits, jnp.float32)


def _stats(F, batch_col, P, CP):
    mean, s2, sk, ku = pl.pallas_call(
        _moments_body,
        grid=(CP // CB,),
        in_specs=[
            pl.BlockSpec((P, CB), lambda c: (0, c)),
            pl.BlockSpec((P, 1), lambda c: (0, 0)),
        ],
        out_specs=[pl.BlockSpec((G, CB), lambda c: (0, c))] * 4,
        out_shape=[jax.ShapeDtypeStruct((G, CP), jnp.float32)] * 4,
        compiler_params=pltpu.CompilerParams(
            dimension_semantics=("arbitrary",)),
    )(F, batch_col)
    med, = pl.pallas_call(
        _median_body,
        grid=(CP // CB, 32),
        in_specs=[
            pl.BlockSpec((P, CB), lambda c, t: (0, c)),
            pl.BlockSpec((P, 1), lambda c, t: (0, 0)),
        ],
        out_specs=[pl.BlockSpec((G, CB), lambda c, t: (0, c))],
        out_shape=[jax.ShapeDtypeStruct((G, CP), jnp.float32)],
        scratch_shapes=[
            pltpu.VMEM((G, CB), jnp.uint32),
            pltpu.VMEM((G, CB), jnp.uint32),
            pltpu.VMEM((G, 1), jnp.float32),
        ],
        compiler_params=pltpu.CompilerParams(
            dimension_semantics=("arbitrary", "arbitrary")),
    )(F, batch_col)
    return mean, med, s2, sk, ku


_NC = 2
_NS = 16
_NW = _NC * _NS


def _zero_ref(ref, n):
    z = jnp.zeros((16,), jnp.float32)

    def bd(i, _):
        ref[pl.ds(i * 16, 16)] = z
        return 0

    lax.fori_loop(0, n // 16, bd, 0)


def _sc_deg(row3, P2, CHN):
    mesh = plsc.VectorSubcoreMesh(core_axis_name="c", subcore_axis_name="s")

    @functools.partial(
        pl.kernel, mesh=mesh,
        compiler_params=pltpu.CompilerParams(needs_layout_passes=False),
        out_type=jax.ShapeDtypeStruct((_NC, P2), jnp.float32),
        scratch_types=[
            pltpu.VMEM((CHN, 128), jnp.int32),
            pltpu.VMEM((128,), jnp.float32),
            pltpu.VMEM((P2,), jnp.float32),
            pltpu.VMEM_SHARED((P2,), jnp.float32),
        ],
    )
    def k(row_hbm, out_hbm, idx_v, ones_v, zbuf_v, shared):
        c = lax.axis_index("c")
        s = lax.axis_index("s")
        wid = s * _NC + c
        pltpu.sync_copy(row_hbm.at[wid], idx_v)
        def ob(i, _):
            ones_v[pl.ds(i * 16, 16)] = jnp.ones((16,), jnp.float32)
            return 0

        lax.fori_loop(0, 8, ob, 0)

        @pl.when(s == 0)
        def _():
            _zero_ref(zbuf_v, P2)
            pltpu.sync_copy(zbuf_v, shared)

        plsc.subcore_barrier()

        def bd(j, _):
            pltpu.sync_copy(ones_v, shared.at[idx_v.at[j]], add=True)
            return 0

        lax.fori_loop(0, CHN, bd, 0)
        plsc.subcore_barrier()

        @pl.when(s == 0)
        def _():
            pltpu.sync_copy(shared, out_hbm.at[c])

    return k(row3)


def _sc_edge_means(row3, col_flat, degp, f4, P2, CHN):
    mesh = plsc.VectorSubcoreMesh(core_axis_name="c", subcore_axis_name="s")

    @functools.partial(
        pl.kernel, mesh=mesh,
        compiler_params=pltpu.CompilerParams(needs_layout_passes=False),
        out_type=[
            jax.ShapeDtypeStruct((_NC, P2), jnp.float32),
            jax.ShapeDtypeStruct((_NC, P2), jnp.float32),
            jax.ShapeDtypeStruct((P2,), jnp.float32),
        ],
        scratch_types=[
            pltpu.VMEM((CHN, 128), jnp.int32),
            pltpu.VMEM((CHN * 128,), jnp.int32),
            pltpu.VMEM((P2,), jnp.float32),
            pltpu.VMEM((P2,), jnp.float32),
            pltpu.VMEM((P2,), jnp.float32),
            pltpu.VMEM((128,), jnp.float32),
            pltpu.VMEM_SHARED((P2,), jnp.float32),
            pltpu.VMEM_SHARED((P2,), jnp.float32),
        ],
    )
    def k(row_hbm, col_hbm, degp_hbm, f4_hbm, fe2_o, f3_o, f1_o,
          idx_v, col_v, dtab, tmp, f1tab, val_v, sh2, sh3):
        c = lax.axis_index("c")
        s = lax.axis_index("s")
        wid = s * _NC + c
        pltpu.sync_copy(row_hbm.at[wid], idx_v)
        pltpu.sync_copy(col_hbm.at[wid], col_v)
        pltpu.sync_copy(degp_hbm.at[0], dtab)
        pltpu.sync_copy(degp_hbm.at[1], tmp)

        def addbd(i, _):
            sl = pl.ds(i * 16, 16)
            dtab[sl] = dtab[sl] + tmp[sl]
            return 0

        lax.fori_loop(0, P2 // 16, addbd, 0)
        pltpu.sync_copy(f4_hbm, tmp)

        def f1bd(i, _):
            sl = pl.ds(i * 16, 16)
            d = dtab[sl]
            sm = d <= 1.0
            den = jnp.where(sm, 1.0, d * (d - 1.0))
            f1tab[sl] = jnp.where(sm, 0.0, (tmp[sl] - d) / den)
            return 0

        lax.fori_loop(0, P2 // 16, f1bd, 0)

        @pl.when((s == 0) & (c == 0))
        def _():
            pltpu.sync_copy(f1tab, f1_o)

        @pl.when(s == 0)
        def _():
            _zero_ref(tmp, P2)
            pltpu.sync_copy(tmp, sh2)
            pltpu.sync_copy(tmp, sh3)

        plsc.subcore_barrier()

        def chunk(j, _):
            def g16d(t, _c):
                ci = col_v[pl.ds(j * 128 + t * 16, 16)]
                val_v[pl.ds(t * 16, 16)] = plsc.load_gather(dtab, [ci])
                return 0

            def g16f(t, _c):
                ci = col_v[pl.ds(j * 128 + t * 16, 16)]
                val_v[pl.ds(t * 16, 16)] = plsc.load_gather(f1tab, [ci])
                return 0

            lax.fori_loop(0, 8, g16d, 0)
            pltpu.sync_copy(val_v, sh2.at[idx_v.at[j]], add=True)
            lax.fori_loop(0, 8, g16f, 0)
            pltpu.sync_copy(val_v, sh3.at[idx_v.at[j]], add=True)
            return 0

        lax.fori_loop(0, CHN, chunk, 0)
        plsc.subcore_barrier()

        @pl.when(s == 0)
        def _():
            pltpu.sync_copy(sh2, fe2_o.at[c])
            pltpu.sync_copy(sh3, f3_o.at[c])

    return k(row3, col_flat, degp, f4)


def kernel(x, edge_index, batch):
    N, DX = x.shape
    P = ((N + BM - 1) // BM) * BM
    P2 = P + 128
    row = edge_index[0].astype(jnp.int32)
    col = edge_index[1].astype(jnp.int32)
    E = row.shape[0]

    CHN = -(-E // (_NW * 128))
    EP = _NW * CHN * 128
    rowp = jnp.full((EP,), P2 - 1, jnp.int32).at[:E].set(row)
    colp = jnp.zeros((EP,), jnp.int32).at[:E].set(col)
    row3 = rowp.reshape(_NW, CHN, 128)
    col2 = colp.reshape(_NW, CHN * 128)

    degp = _sc_deg(row3, P2, CHN)
    deg_row = (degp[0:1, :P] + degp[1:2, :P])
    deg = deg_row[0, :N]

    Mf = jnp.zeros((P, P), jnp.float32).at[row, col].add(1.0)
    Mb, Ab = _prep(Mf, N, P)

    f4p, f5rp, f6rp = _ego(Ab, Mb, deg_row, P)
    f4 = f4p[:N, 0]
    f5r = f5rp[:N, 0]
    f6r = f6rp[:N, 0]

    f4c = jnp.zeros((P2,), jnp.float32).at[:P].set(f4p[:, 0])
    fe2n, f3n, f1p = _sc_edge_means(row3, col2, degp, f4c, P2, CHN)
    cmax = jnp.maximum(deg, 1.0)
    f1 = f1p[:N]
    feat2 = (fe2n[0, :N] + fe2n[1, :N]) / cmax
    f3 = (f3n[0, :N] + f3n[1, :N]) / cmax
    f5f = f5r - 2.0 * f4
    f6f = f6r - deg - 1.0

    topo = jnp.stack([deg, f1, feat2, f3, f4, f5f, f6f], axis=1)
    F = jnp.concatenate([x, topo], axis=1)
    C = DX + 7
    CP = ((C + CB - 1) // CB) * CB
    Fp = jnp.zeros((P, CP), jnp.float32).at[:N, :C].set(F)
    btp = jnp.full((P, 1), 255, jnp.int32).at[:N, 0].set(batch.astype(jnp.int32))

    mean, med, s2, sk, ku = _stats(Fp, btp, P, CP)
    return jnp.hstack([mean[:, :C], med[:, :C], s2[:, :C],
                       sk[:, :C], ku[:, :C]])
